# baseline (speedup 1.0000x reference)
import functools

import jax
import jax.numpy as jnp
from jax.experimental import pallas as pl
from jax.experimental.pallas import tpu as pltpu


def _enc_select_kernel(x_ref, wenc_ref, benc_ref, dbias_ref, out_ref,
                       vb0, vb1, tbits_ref, cnt_ref, *, k, n_i, n_j):
    i = pl.program_id(0)
    j = pl.program_id(1)
    jb = wenc_ref.shape[0]
    b_rows = out_ref.shape[0]
    d_sae = jb * n_j
    nl = 512
    n_ch = d_sae // nl
    mw = d_sae // 8

    @pl.when(i < n_i)
    def _matmul():
        xb = x_ref[...] - dbias_ref[...]
        pre = jax.lax.dot_general(
            xb, wenc_ref[...], (((1,), (1,)), ((), ())),
            preferred_element_type=jnp.float32,
        ) + benc_ref[...]
        v = jnp.maximum(pre, 0.0)

        @pl.when(i % 2 == 0)
        def _():
            vb0[:, pl.ds(j * jb, jb)] = v

        @pl.when(i % 2 == 1)
        def _():
            vb1[:, pl.ds(j * jb, jb)] = v

    @pl.when(i > 0)
    def _select():
        @pl.when(j == 0)
        def _init():
            tbits_ref[...] = jnp.zeros((b_rows, 1), jnp.int32)
            cnt_ref[...] = jnp.full((b_rows, 1), -1, jnp.int32)

        def count_ge(vref, cand_f):
            acc = jnp.zeros((b_rows, nl), jnp.int32)
            for c in range(n_ch):
                acc += (vref[:, c * nl:(c + 1) * nl] >= cand_f).astype(jnp.int32)
            return jnp.sum(acc, axis=1, keepdims=True)

        def bit_round(vref, bi):
            t = tbits_ref[...]
            cand = t | jnp.left_shift(1, 30 - bi)
            cand_f = jax.lax.bitcast_convert_type(cand, jnp.float32)
            c_ = count_ge(vref, cand_f)
            take = c_ >= k
            tbits_ref[...] = jnp.where(take, cand, t)
            cnt_ref[...] = jnp.where(take, c_, cnt_ref[...])

        def do_phase(vref):
            @pl.when(j < 8)
            def _a():
                bit_round(vref, 2 * j)
                bit_round(vref, 2 * j + 1)

            @pl.when(jnp.logical_and(j >= 8, j < 23))
            def _b():
                bit_round(vref, j + 8)

            @pl.when(j >= n_j - 8)
            def _m():
                t_f = jax.lax.bitcast_convert_type(tbits_ref[...], jnp.float32)
                c0 = (j - (n_j - 8)) * mw
                blk = vref[:, pl.ds(c0, mw)]
                out_ref[...] = jnp.where(blk >= t_f, blk, 0.0)

        @pl.when(i % 2 == 1)
        def _p0():
            do_phase(vb0)

        @pl.when(i % 2 == 0)
        def _p1():
            do_phase(vb1)


def _decode_kernel(lat_ref, wdec_ref, dbias_ref, y_ref):
    kstep = pl.program_id(1)
    lat = lat_ref[...].astype(jnp.bfloat16)
    acc = jax.lax.dot_general(
        lat, wdec_ref[...], (((1,), (1,)), ((), ())),
        preferred_element_type=jnp.float32,
    )

    @pl.when(kstep == 0)
    def _():
        y_ref[...] = acc + dbias_ref[...]

    @pl.when(kstep != 0)
    def _():
        y_ref[...] += acc


def _forward(x, W_enc, b_enc2, W_dec, dbias2):
    n_tok, d_model = x.shape
    d_sae = W_enc.shape[0]
    k = 100

    bt = min(256, n_tok)
    n_j = 32
    jb = d_sae // n_j
    n_i = n_tok // bt

    latents = pl.pallas_call(
        functools.partial(_enc_select_kernel, k=k, n_i=n_i, n_j=n_j),
        grid=(n_i + 1, n_j),
        in_specs=[
            pl.BlockSpec((bt, d_model),
                         lambda i, j: (jnp.minimum(i, n_i - 1), 0)),
            pl.BlockSpec((jb, d_model),
                         lambda i, j: (jnp.where(i < n_i, j, n_j - 1), 0)),
            pl.BlockSpec((1, jb),
                         lambda i, j: (0, jnp.where(i < n_i, j, n_j - 1))),
            pl.BlockSpec((1, d_model), lambda i, j: (0, 0)),
        ],
        out_specs=pl.BlockSpec(
            (bt, d_sae // 8),
            lambda i, j: (jnp.maximum(i - 1, 0),
                          jnp.where(i == 0, 0,
                                    jnp.maximum(j - (n_j - 8), 0)))),
        out_shape=jax.ShapeDtypeStruct((n_tok, d_sae), jnp.float32),
        scratch_shapes=[
            pltpu.VMEM((bt, d_sae), jnp.float32),
            pltpu.VMEM((bt, d_sae), jnp.float32),
            pltpu.VMEM((bt, 1), jnp.int32),
            pltpu.VMEM((bt, 1), jnp.int32),
        ],
        compiler_params=pltpu.CompilerParams(
            dimension_semantics=("arbitrary", "arbitrary"),
        ),
    )(x, W_enc, b_enc2, dbias2)

    bt2 = min(1024, n_tok)
    kb = min(1024, d_sae)
    n_i2, n_k = n_tok // bt2, d_sae // kb

    y = pl.pallas_call(
        _decode_kernel,
        grid=(n_i2, n_k),
        in_specs=[
            pl.BlockSpec((bt2, kb), lambda i, kk: (i, kk)),
            pl.BlockSpec((d_model, kb), lambda i, kk: (0, kk)),
            pl.BlockSpec((1, d_model), lambda i, kk: (0, 0)),
        ],
        out_specs=pl.BlockSpec((bt2, d_model), lambda i, kk: (i, 0)),
        out_shape=jax.ShapeDtypeStruct((n_tok, d_model), jnp.float32),
        compiler_params=pltpu.CompilerParams(
            dimension_semantics=("parallel", "arbitrary"),
        ),
    )(latents, W_dec, dbias2)

    return (y, latents)


def kernel(x, W_enc, b_enc, W_dec, dec_bias):
    n_tok, d_model = x.shape
    d_sae = W_enc.shape[0]
    b_enc2 = b_enc.reshape(1, d_sae)
    dbias2 = dec_bias.reshape(1, d_model)
    wdec16 = W_dec.astype(jnp.bfloat16)

    devs = jax.devices()
    n_dev = len(devs)
    while n_dev > 1 and n_tok % n_dev:
        n_dev -= 1
    if n_dev == 1:
        return _forward(x, W_enc, b_enc2, wdec16, dbias2)

    mesh = jax.sharding.Mesh(devs[:n_dev], ("d",))
    P = jax.sharding.PartitionSpec
    fwd = jax.shard_map(
        _forward, mesh=mesh,
        in_specs=(P("d", None), P(None, None), P(None, None),
                  P(None, None), P(None, None)),
        out_specs=(P("d", None), P("d", None)),
        check_vma=False,
    )
    return fwd(x, W_enc, b_enc2, wdec16, dbias2)


# speedup vs baseline: 1.1306x; 1.1306x over previous
import functools

import jax
import jax.numpy as jnp
from jax.experimental import pallas as pl
from jax.experimental.pallas import tpu as pltpu


def _enc_select_kernel(x_ref, wenc_ref, benc_ref, dbias_ref, out_ref, vc_ref, *, k):
    j = pl.program_id(1)
    nj = pl.num_programs(1)
    jb = wenc_ref.shape[0]

    xb = x_ref[...] - dbias_ref[...]
    pre = jax.lax.dot_general(
        xb, wenc_ref[...], (((1,), (1,)), ((), ())),
        preferred_element_type=jnp.float32,
    ) + benc_ref[...]
    v = jnp.maximum(pre, 0.0)
    out_ref[:, pl.ds(j * jb, jb)] = v
    u = jax.lax.bitcast_convert_type(v, jnp.int32)
    vt = jax.lax.bitcast_convert_type(u & jnp.int32(-65536), jnp.float32)
    vc_ref[:, pl.ds(j * jb, jb)] = vt.astype(jnp.bfloat16)

    @pl.when(j == nj - 1)
    def _select():
        b_rows, d_sae = out_ref.shape
        nl = 128
        n_ch = d_sae // nl
        nl16 = 256
        n_ch16 = d_sae // nl16

        def count16(cand_bits):
            cand_bf = jax.lax.bitcast_convert_type(
                cand_bits, jnp.float32).astype(jnp.bfloat16)
            acc = jnp.zeros((b_rows, nl16), jnp.bfloat16)
            one = jnp.ones((b_rows, nl16), jnp.bfloat16)
            zero = jnp.zeros((b_rows, nl16), jnp.bfloat16)
            for c in range(n_ch16):
                m = vc_ref[:, c * nl16:(c + 1) * nl16] >= cand_bf
                acc += jnp.where(m, one, zero)
            return jnp.sum(acc.astype(jnp.float32), axis=1,
                           keepdims=True).astype(jnp.int32)

        def count_ge(cand_f):
            acc = jnp.zeros((b_rows, nl), jnp.int32)
            for c in range(n_ch):
                acc += (out_ref[:, c * nl:(c + 1) * nl] >= cand_f).astype(jnp.int32)
            return jnp.sum(acc, axis=1, keepdims=True)

        def cond_a(st):
            i, _, cnt = st
            return jnp.logical_and(i < 15, jnp.logical_not(jnp.all(cnt == k)))

        def body_a(st):
            i, t, cnt = st
            cand = t | jnp.left_shift(1, 30 - i)
            c = count16(cand)
            take = c >= k
            return (i + 1, jnp.where(take, cand, t), jnp.where(take, c, cnt))

        st_a = jax.lax.while_loop(
            cond_a, body_a,
            (jnp.int32(0), jnp.zeros((b_rows, 1), jnp.int32),
             jnp.full((b_rows, 1), -1, jnp.int32)))

        def cond_b(st):
            i, _, cnt = st
            return jnp.logical_and(i < 31, jnp.logical_not(jnp.all(cnt == k)))

        def body_b(st):
            i, t, cnt = st
            cand = t | jnp.left_shift(1, 30 - i)
            cand_f = jax.lax.bitcast_convert_type(cand, jnp.float32)
            c = count_ge(cand_f)
            take = c >= k
            return (i + 1, jnp.where(take, cand, t), jnp.where(take, c, cnt))

        _, t_bits, _ = jax.lax.while_loop(
            cond_b, body_b, (jnp.int32(15), st_a[1], st_a[2]))
        t_f = jax.lax.bitcast_convert_type(t_bits, jnp.float32)

        for c in range(n_ch):
            blk = out_ref[:, c * nl:(c + 1) * nl]
            out_ref[:, c * nl:(c + 1) * nl] = jnp.where(blk >= t_f, blk, 0.0)


def _decode_kernel(lat_ref, wdec_ref, dbias_ref, y_ref):
    kstep = pl.program_id(1)
    lat = lat_ref[...].astype(jnp.bfloat16)
    acc = jax.lax.dot_general(
        lat, wdec_ref[...], (((1,), (1,)), ((), ())),
        preferred_element_type=jnp.float32,
    )

    @pl.when(kstep == 0)
    def _():
        y_ref[...] = acc + dbias_ref[...]

    @pl.when(kstep != 0)
    def _():
        y_ref[...] += acc


def _forward(x, W_enc, b_enc2, W_dec, dbias2):
    n_tok, d_model = x.shape
    d_sae = W_enc.shape[0]
    k = 100

    bt = min(256, n_tok)
    jb = min(512, d_sae)
    n_i, n_j = n_tok // bt, d_sae // jb

    latents = pl.pallas_call(
        functools.partial(_enc_select_kernel, k=k),
        grid=(n_i, n_j),
        in_specs=[
            pl.BlockSpec((bt, d_model), lambda i, j: (i, 0)),
            pl.BlockSpec((jb, d_model), lambda i, j: (j, 0)),
            pl.BlockSpec((1, jb), lambda i, j: (0, j)),
            pl.BlockSpec((1, d_model), lambda i, j: (0, 0)),
        ],
        out_specs=pl.BlockSpec((bt, d_sae), lambda i, j: (i, 0)),
        out_shape=jax.ShapeDtypeStruct((n_tok, d_sae), jnp.float32),
        scratch_shapes=[pltpu.VMEM((bt, d_sae), jnp.bfloat16)],
        compiler_params=pltpu.CompilerParams(
            dimension_semantics=("parallel", "arbitrary"),
        ),
    )(x, W_enc, b_enc2, dbias2)

    bt2 = min(1024, n_tok)
    kb = min(2048, d_sae)
    n_i2, n_k = n_tok // bt2, d_sae // kb
    wdec16 = W_dec

    y = pl.pallas_call(
        _decode_kernel,
        grid=(n_i2, n_k),
        in_specs=[
            pl.BlockSpec((bt2, kb), lambda i, kk: (i, kk)),
            pl.BlockSpec((d_model, kb), lambda i, kk: (0, kk)),
            pl.BlockSpec((1, d_model), lambda i, kk: (0, 0)),
        ],
        out_specs=pl.BlockSpec((bt2, d_model), lambda i, kk: (i, 0)),
        out_shape=jax.ShapeDtypeStruct((n_tok, d_model), jnp.float32),
        compiler_params=pltpu.CompilerParams(
            dimension_semantics=("parallel", "arbitrary"),
        ),
    )(latents, wdec16, dbias2)

    return (y, latents)


def kernel(x, W_enc, b_enc, W_dec, dec_bias):
    n_tok, d_model = x.shape
    d_sae = W_enc.shape[0]
    b_enc2 = b_enc.reshape(1, d_sae)
    dbias2 = dec_bias.reshape(1, d_model)
    wdec16 = W_dec.astype(jnp.bfloat16)

    devs = jax.devices()
    n_dev = len(devs)
    while n_dev > 1 and n_tok % n_dev:
        n_dev -= 1
    if n_dev == 1:
        return _forward(x, W_enc, b_enc2, wdec16, dbias2)

    mesh = jax.sharding.Mesh(devs[:n_dev], ("d",))
    P = jax.sharding.PartitionSpec
    fwd = jax.shard_map(
        _forward, mesh=mesh,
        in_specs=(P("d", None), P(None, None), P(None, None),
                  P(None, None), P(None, None)),
        out_specs=(P("d", None), P("d", None)),
        check_vma=False,
    )
    return fwd(x, W_enc, b_enc2, wdec16, dbias2)
